# baseline (speedup 1.0000x reference)
import jax
import jax.numpy as jnp
from jax.experimental import pallas as pl
from jax.experimental.pallas import tpu as pltpu

_SEG = 2048
_CS = 512


def _tc_body(starts_ref, a_ref, b_ref, out_ref):
    b = pl.program_id(0)
    st = starts_ref[b]
    sm = st % _CS
    win = jnp.concatenate([a_ref[0], b_ref[0]], axis=0)
    rolled = pltpu.roll(win, 2 * _CS - sm, 0)
    out_ref[0] = rolled[0:_CS]


def kernel(patterns, segment_size, lengths, offsets):
    B, T, D = patterns.shape
    S = _SEG
    NC = S // _CS
    NBLK = T // _CS
    starts = jnp.clip(
        offsets.astype(jnp.int32)
        + (jnp.asarray(segment_size, jnp.int32) - jnp.int32(S)),
        0,
        T - S,
    )
    segments = pl.pallas_call(
        _tc_body,
        grid_spec=pltpu.PrefetchScalarGridSpec(
            num_scalar_prefetch=1,
            grid=(B, NC),
            in_specs=[
                pl.BlockSpec(
                    (1, _CS, D),
                    lambda b, c, st: (b, st[b] // _CS + c, 0),
                ),
                pl.BlockSpec(
                    (1, _CS, D),
                    lambda b, c, st: (
                        b,
                        jnp.minimum(st[b] // _CS + c + 1, NBLK - 1),
                        0,
                    ),
                ),
            ],
            out_specs=pl.BlockSpec((1, _CS, D), lambda b, c, st: (b, c, 0)),
        ),
        out_shape=jax.ShapeDtypeStruct((B, S, D), patterns.dtype),
    )(starts, patterns, patterns)
    return (segments, offsets)


# speedup vs baseline: 1.6570x; 1.6570x over previous
import functools

import jax
import jax.numpy as jnp
from jax import lax
from jax.experimental import pallas as pl
from jax.experimental.pallas import tpu as pltpu
from jax.experimental.pallas import tpu_sc as plsc

_SEG = 2048
_CH = 32
_NBUF = 4


def kernel(patterns, segment_size, lengths, offsets):
    B, T, D = patterns.shape
    S = _SEG
    starts = jnp.clip(
        offsets.astype(jnp.int32)
        + (jnp.asarray(segment_size, jnp.int32) - jnp.int32(S)),
        0,
        T - S,
    )
    flat_in = patterns.reshape(B * T, D)
    mesh = plsc.VectorSubcoreMesh(core_axis_name="c", subcore_axis_name="s")
    NW = 32
    ROWS = (B * S) // NW
    PER_B = S // ROWS
    ITERS = ROWS // _CH
    ROUNDS = ITERS // _NBUF

    @functools.partial(
        pl.kernel,
        out_type=jax.ShapeDtypeStruct((B * S, D), patterns.dtype),
        mesh=mesh,
        scratch_types=[
            pltpu.VMEM((16,), jnp.int32),
            [pltpu.VMEM((_CH, D), jnp.float32) for _ in range(_NBUF)],
            [pltpu.VMEM((_CH,), jnp.int32) for _ in range(_NBUF)],
            [pltpu.SemaphoreType.DMA for _ in range(_NBUF)],
            [pltpu.SemaphoreType.DMA for _ in range(_NBUF)],
        ],
    )
    def run(in_hbm, starts_hbm, out_hbm, st_v, bufs, idxs, gsems, ssems):
        c = lax.axis_index("c")
        s = lax.axis_index("s")
        wid = s * 2 + c
        b = wid // PER_B
        h = wid % PER_B
        pltpu.sync_copy(starts_hbm, st_v)
        stv = st_v[...]
        lane = lax.iota(jnp.int32, 16)
        start_vec = lax.gather(
            stv,
            jnp.full((16, 1), b, jnp.int32),
            lax.GatherDimensionNumbers(
                offset_dims=(), collapsed_slice_dims=(0,), start_index_map=(0,)
            ),
            (1,),
            mode=lax.GatherScatterMode.PROMISE_IN_BOUNDS,
        )
        src0_vec = start_vec + (b * T + h * ROWS) + lane
        dst0 = wid * ROWS

        def fill_idx(i, k):
            for j in range(_CH // 16):
                idxs[k][pl.ds(j * 16, 16)] = src0_vec + (i * _CH + j * 16)

        def gather(k):
            return pltpu.make_async_copy(in_hbm.at[idxs[k]], bufs[k], gsems[k])

        def store(i, k):
            d0 = pl.multiple_of(dst0 + i * _CH, 8)
            return pltpu.make_async_copy(
                bufs[k], out_hbm.at[pl.ds(d0, _CH), :], ssems[k]
            )

        for k in range(_NBUF):
            fill_idx(k, k)
            gather(k).start()

        def body(g, carry):
            for k in range(_NBUF):
                i = g * _NBUF + k
                gather(k).wait()
                store(i, k).start()

                @pl.when(i + _NBUF < ITERS)
                def _():
                    store(i, k).wait()
                    fill_idx(i + _NBUF, k)
                    gather(k).start()

            return carry

        lax.fori_loop(0, ROUNDS, body, 0)
        for k in range(_NBUF):
            store(ITERS - _NBUF + k, k).wait()

    out = run(flat_in, starts)
    return (out.reshape(B, S, D), offsets)


# speedup vs baseline: 1.6728x; 1.0095x over previous
import functools

import jax
import jax.numpy as jnp
from jax import lax
from jax.experimental import pallas as pl
from jax.experimental.pallas import tpu as pltpu
from jax.experimental.pallas import tpu_sc as plsc

_SEG = 2048
_CH = 16
_NBUF = 8


def kernel(patterns, segment_size, lengths, offsets):
    B, T, D = patterns.shape
    S = _SEG
    starts = jnp.clip(
        offsets.astype(jnp.int32)
        + (jnp.asarray(segment_size, jnp.int32) - jnp.int32(S)),
        0,
        T - S,
    )
    flat_in = patterns.reshape(B * T, D)
    mesh = plsc.VectorSubcoreMesh(core_axis_name="c", subcore_axis_name="s")
    NW = 32
    ROWS = (B * S) // NW
    PER_B = S // ROWS
    ITERS = ROWS // _CH
    ROUNDS = ITERS // _NBUF

    @functools.partial(
        pl.kernel,
        out_type=jax.ShapeDtypeStruct((B * S, D), patterns.dtype),
        mesh=mesh,
        scratch_types=[
            pltpu.VMEM((16,), jnp.int32),
            [pltpu.VMEM((_CH, D), jnp.float32) for _ in range(_NBUF)],
            [pltpu.VMEM((_CH,), jnp.int32) for _ in range(_NBUF)],
            [pltpu.SemaphoreType.DMA for _ in range(_NBUF)],
            [pltpu.SemaphoreType.DMA for _ in range(_NBUF)],
        ],
    )
    def run(in_hbm, starts_hbm, out_hbm, st_v, bufs, idxs, gsems, ssems):
        c = lax.axis_index("c")
        s = lax.axis_index("s")
        wid = s * 2 + c
        b = wid // PER_B
        h = wid % PER_B
        pltpu.sync_copy(starts_hbm, st_v)
        stv = st_v[...]
        lane = lax.iota(jnp.int32, 16)
        start_vec = lax.gather(
            stv,
            jnp.full((16, 1), b, jnp.int32),
            lax.GatherDimensionNumbers(
                offset_dims=(), collapsed_slice_dims=(0,), start_index_map=(0,)
            ),
            (1,),
            mode=lax.GatherScatterMode.PROMISE_IN_BOUNDS,
        )
        src0_vec = start_vec + (b * T + h * ROWS) + lane
        dst0 = wid * ROWS

        def fill_idx(i, k):
            for j in range(_CH // 16):
                idxs[k][pl.ds(j * 16, 16)] = src0_vec + (i * _CH + j * 16)

        def gather(k):
            return pltpu.make_async_copy(in_hbm.at[idxs[k]], bufs[k], gsems[k])

        def store(i, k):
            d0 = pl.multiple_of(dst0 + i * _CH, 8)
            return pltpu.make_async_copy(
                bufs[k], out_hbm.at[pl.ds(d0, _CH), :], ssems[k]
            )

        for k in range(_NBUF):
            fill_idx(k, k)
            gather(k).start()

        def body(g, carry):
            for k in range(_NBUF):
                i = g * _NBUF + k
                gather(k).wait()
                store(i, k).start()

                @pl.when(i + _NBUF < ITERS)
                def _():
                    store(i, k).wait()
                    fill_idx(i + _NBUF, k)
                    gather(k).start()

            return carry

        lax.fori_loop(0, ROUNDS, body, 0)
        for k in range(_NBUF):
            store(ITERS - _NBUF + k, k).wait()

    out = run(flat_in, starts)
    return (out.reshape(B, S, D), offsets)
